# Initial kernel scaffold; baseline (speedup 1.0000x reference)
import functools

import jax
import jax.numpy as jnp
from jax import lax
from jax.experimental import pallas as pl
from jax.experimental.pallas import tpu as pltpu
from jax.experimental.pallas import tpu_sc as plsc

N_NODES = 10000
N_EDGES = 320000
D = 128

NC = 2
NS = 16
NW = NC * NS
L = 16

CHUNK = 128
N_CHUNKS = N_EDGES // CHUNK
CHUNKS_PER_W = N_CHUNKS // NW
CHUNKS_REM = N_CHUNKS % NW
ROWS_PER_TILE = N_NODES // NS
ZROWS = 125


def _lane_broadcast(v, lane):
    return lax.broadcast_in_dim(v[lane], (L,), ())


def _linear_body(x_ref, wt_ref, b_ref, o_ref):
    o_ref[...] = (
        jnp.dot(x_ref[...], wt_ref[...], preferred_element_type=jnp.float32,
                precision=lax.Precision.HIGHEST)
        + b_ref[...]
    )


def _combine_body(p_ref, o_ref):
    o_ref[...] = p_ref[0] + p_ref[1]


def _sc_body(src_hbm, dst_hbm, w_hbm, support_hbm, out_hbm,
             src_v, dst_v, w_v, rows_v, acc_sh, sem):
    ci = lax.axis_index("c")
    si = lax.axis_index("s")
    wid = si * NC + ci

    zero16 = jnp.zeros((L,), jnp.float32)

    def zrow(r, carry):
        for j in range(D // L):
            rows_v[r, j * L:(j + 1) * L] = zero16
        return carry

    lax.fori_loop(0, ZROWS, zrow, 0)
    base = si * ROWS_PER_TILE
    for t in range(ROWS_PER_TILE // ZROWS):
        pltpu.sync_copy(rows_v.at[pl.ds(0, ZROWS)],
                        acc_sh.at[pl.ds(base + t * ZROWS, ZROWS)])
    plsc.subcore_barrier()

    def chunk_body(i, carry):
        c = wid + i * NW
        e0 = c * CHUNK
        pltpu.sync_copy(src_hbm.at[pl.ds(e0, CHUNK)], src_v)
        pltpu.sync_copy(dst_hbm.at[pl.ds(e0, CHUNK)], dst_v)
        pltpu.sync_copy(w_hbm.at[pl.ds(e0, CHUNK)], w_v)
        pltpu.async_copy(support_hbm.at[src_v], rows_v, sem).wait()

        def grp(g, gc):
            wv = w_v[pl.ds(g * L, L)]
            for r in range(L):
                wb = _lane_broadcast(wv, r)
                row = g * L + r
                for j in range(D // L):
                    sl = pl.ds(j * L, L)
                    rows_v[row, sl] = rows_v[row, sl] * wb
            return gc

        lax.fori_loop(0, CHUNK // L, grp, 0)
        pltpu.sync_copy(rows_v, acc_sh.at[dst_v], add=True)
        return carry

    n_my = CHUNKS_PER_W + jnp.where(wid < CHUNKS_REM, 1, 0)
    lax.fori_loop(0, n_my, chunk_body, 0)
    plsc.subcore_barrier()

    pltpu.sync_copy(acc_sh.at[pl.ds(base, ROWS_PER_TILE)],
                    out_hbm.at[ci, pl.ds(base, ROWS_PER_TILE)])


_sc_call = functools.partial(
    pl.kernel,
    out_type=jax.ShapeDtypeStruct((NC, N_NODES, D), jnp.float32),
    mesh=plsc.VectorSubcoreMesh(core_axis_name="c", subcore_axis_name="s"),
    scratch_types=[
        pltpu.VMEM((CHUNK,), jnp.int32),
        pltpu.VMEM((CHUNK,), jnp.int32),
        pltpu.VMEM((CHUNK,), jnp.float32),
        pltpu.VMEM((CHUNK, D), jnp.float32),
        pltpu.VMEM_SHARED((N_NODES, D), jnp.float32),
        pltpu.SemaphoreType.DMA,
    ],
)(_sc_body)

_ROWS_BLK = 1000


def kernel(x, edge_index, edge_weight, W, b):
    src = edge_index[0]
    dst = edge_index[1]
    wt = W.T
    b2 = b.reshape(1, D)

    support = pl.pallas_call(
        _linear_body,
        grid=(N_NODES // _ROWS_BLK,),
        in_specs=[
            pl.BlockSpec((_ROWS_BLK, D), lambda i: (i, 0)),
            pl.BlockSpec((D, D), lambda i: (0, 0)),
            pl.BlockSpec((1, D), lambda i: (0, 0)),
        ],
        out_specs=pl.BlockSpec((_ROWS_BLK, D), lambda i: (i, 0)),
        out_shape=jax.ShapeDtypeStruct((N_NODES, D), jnp.float32),
    )(x, wt, b2)

    partials = _sc_call(src, dst, edge_weight, support)

    out = pl.pallas_call(
        _combine_body,
        grid=(N_NODES // _ROWS_BLK,),
        in_specs=[pl.BlockSpec((NC, _ROWS_BLK, D), lambda i: (0, i, 0))],
        out_specs=pl.BlockSpec((_ROWS_BLK, D), lambda i: (i, 0)),
        out_shape=jax.ShapeDtypeStruct((N_NODES, D), jnp.float32),
    )(partials)

    return out


# speedup vs baseline: 4.9272x; 4.9272x over previous
import functools

import jax
import jax.numpy as jnp
from jax import lax
from jax.experimental import pallas as pl
from jax.experimental.pallas import tpu as pltpu
from jax.experimental.pallas import tpu_sc as plsc

N_NODES = 10000
N_EDGES = 320000
D = 128

NC = 2
NS = 16
NW = NC * NS
L = 16

CHUNK = 128
N_CHUNKS = N_EDGES // CHUNK
CHUNKS_PER_W = N_CHUNKS // NW
CHUNKS_REM = N_CHUNKS % NW
N_ACC = 10240
STRIPE = N_ACC // NS
LAST_STRIPE = N_NODES - (NS - 1) * STRIPE


def _lane_broadcast(v, lane):
    return lax.broadcast_in_dim(v[lane], (L,), ())


def _linear_body(x_ref, wt_ref, b_ref, o_ref):
    o_ref[...] = (
        jnp.dot(x_ref[...], wt_ref[...], preferred_element_type=jnp.float32,
                precision=lax.Precision.HIGHEST)
        + b_ref[...]
    )


def _combine_body(p_ref, o_ref):
    o_ref[...] = p_ref[0] + p_ref[1]


def _sc_body(src_hbm, dst_hbm, w_hbm, support_hbm, out_hbm,
             src_v, dst_v, w_v, rows_v, acc_sh, sem):
    ci = lax.axis_index("c")
    si = lax.axis_index("s")
    wid = si * NC + ci

    zero16 = jnp.zeros((L,), jnp.float32)

    def zrow(r, carry):
        for j in range(D // L):
            rows_v[r, j * L:(j + 1) * L] = zero16
        return carry

    lax.fori_loop(0, CHUNK, zrow, 0)
    base = si * STRIPE
    for t in range(STRIPE // CHUNK):
        pltpu.sync_copy(rows_v, acc_sh.at[pl.ds(base + t * CHUNK, CHUNK)])
    plsc.subcore_barrier()

    def chunk_body(i, carry):
        c = wid + i * NW
        e0 = c * CHUNK
        pltpu.sync_copy(src_hbm.at[pl.ds(e0, CHUNK)], src_v)
        pltpu.sync_copy(dst_hbm.at[pl.ds(e0, CHUNK)], dst_v)
        pltpu.sync_copy(w_hbm.at[pl.ds(e0, CHUNK)], w_v)
        pltpu.async_copy(support_hbm.at[src_v], rows_v, sem).wait()

        def grp(g, gc):
            wv = w_v[pl.ds(g * L, L)]
            for r in range(L):
                wb = _lane_broadcast(wv, r)
                row = g * L + r
                for j in range(D // L):
                    sl = pl.ds(j * L, L)
                    rows_v[row, sl] = rows_v[row, sl] * wb
            return gc

        lax.fori_loop(0, CHUNK // L, grp, 0)
        pltpu.sync_copy(rows_v, acc_sh.at[dst_v], add=True)
        return carry

    n_my = CHUNKS_PER_W + jnp.where(wid < CHUNKS_REM, 1, 0)
    lax.fori_loop(0, n_my, chunk_body, 0)
    plsc.subcore_barrier()

    @pl.when(si < NS - 1)
    def _():
        pltpu.sync_copy(acc_sh.at[pl.ds(base, STRIPE)],
                        out_hbm.at[ci, pl.ds(base, STRIPE)])

    @pl.when(si == NS - 1)
    def _():
        pltpu.sync_copy(acc_sh.at[pl.ds(base, LAST_STRIPE)],
                        out_hbm.at[ci, pl.ds(base, LAST_STRIPE)])


_sc_call = functools.partial(
    pl.kernel,
    out_type=jax.ShapeDtypeStruct((NC, N_NODES, D), jnp.float32),
    mesh=plsc.VectorSubcoreMesh(core_axis_name="c", subcore_axis_name="s"),
    scratch_types=[
        pltpu.VMEM((CHUNK,), jnp.int32),
        pltpu.VMEM((CHUNK,), jnp.int32),
        pltpu.VMEM((CHUNK,), jnp.float32),
        pltpu.VMEM((CHUNK, D), jnp.float32),
        pltpu.VMEM_SHARED((N_ACC, D), jnp.float32),
        pltpu.SemaphoreType.DMA,
    ],
)(_sc_body)

_ROWS_BLK = 1000


def kernel(x, edge_index, edge_weight, W, b):
    src = edge_index[0]
    dst = edge_index[1]
    wt = W.T
    b2 = b.reshape(1, D)

    support = pl.pallas_call(
        _linear_body,
        grid=(N_NODES // _ROWS_BLK,),
        in_specs=[
            pl.BlockSpec((_ROWS_BLK, D), lambda i: (i, 0)),
            pl.BlockSpec((D, D), lambda i: (0, 0)),
            pl.BlockSpec((1, D), lambda i: (0, 0)),
        ],
        out_specs=pl.BlockSpec((_ROWS_BLK, D), lambda i: (i, 0)),
        out_shape=jax.ShapeDtypeStruct((N_NODES, D), jnp.float32),
    )(x, wt, b2)

    partials = _sc_call(src, dst, edge_weight, support)

    out = pl.pallas_call(
        _combine_body,
        grid=(N_NODES // _ROWS_BLK,),
        in_specs=[pl.BlockSpec((NC, _ROWS_BLK, D), lambda i: (0, i, 0))],
        out_specs=pl.BlockSpec((_ROWS_BLK, D), lambda i: (i, 0)),
        out_shape=jax.ShapeDtypeStruct((N_NODES, D), jnp.float32),
    )(partials)

    return out
